# Initial kernel scaffold; baseline (speedup 1.0000x reference)
import jax
import jax.numpy as jnp
from jax.experimental import pallas as pl


def kernel(x, x_len, embed):
    raise NotImplementedError("write your pallas kernel here")


# speedup vs baseline: 2.2752x; 2.2752x over previous
import jax
import jax.numpy as jnp
from jax.experimental import pallas as pl
from jax.experimental.pallas import tpu as pltpu

_B, _T, _D = 16, 576, 256
_K = 1024
_N = _B * _T
_BN = 512


def _body(x_ref, et_ref, e_ref, q_ref, ind_ref, dist_ref):
    x = x_ref[...]
    et = et_ref[...]
    xe = jax.lax.dot_general(
        x, et, (((1,), (0,)), ((), ())),
        preferred_element_type=jnp.float32)
    x2 = jnp.sum(x * x, axis=1, keepdims=True)
    e2 = jnp.sum(et * et, axis=0, keepdims=True)
    dist = -(x2 - 2.0 * xe + e2)
    dist_ref[...] = dist
    m = jnp.max(dist, axis=1, keepdims=True)
    iota_k = jax.lax.broadcasted_iota(jnp.int32, dist.shape, 1)
    ind = jnp.min(jnp.where(dist == m, iota_k, _K), axis=1, keepdims=True)
    ind_ref[...] = ind
    onehot = (iota_k == ind).astype(jnp.float32)
    q_ref[...] = jax.lax.dot_general(
        onehot, e_ref[...], (((1,), (0,)), ((), ())),
        preferred_element_type=jnp.float32)


def kernel(x, x_len, embed):
    del x_len
    xf = x.reshape(_N, _D)
    embed_t = embed.T
    grid = (_N // _BN,)
    q, ind, dist = pl.pallas_call(
        _body,
        grid=grid,
        in_specs=[
            pl.BlockSpec((_BN, _D), lambda i: (i, 0)),
            pl.BlockSpec((_D, _K), lambda i: (0, 0)),
            pl.BlockSpec((_K, _D), lambda i: (0, 0)),
        ],
        out_specs=[
            pl.BlockSpec((_BN, _D), lambda i: (i, 0)),
            pl.BlockSpec((_BN, 1), lambda i: (i, 0)),
            pl.BlockSpec((_BN, _K), lambda i: (i, 0)),
        ],
        out_shape=[
            jax.ShapeDtypeStruct((_N, _D), jnp.float32),
            jax.ShapeDtypeStruct((_N, 1), jnp.int32),
            jax.ShapeDtypeStruct((_N, _K), jnp.float32),
        ],
        compiler_params=pltpu.CompilerParams(
            dimension_semantics=("parallel",),
        ),
    )(xf, embed_t, embed)
    return (q.reshape(_B, _T, _D), ind.reshape(_B, _T), dist.reshape(_B, _T, _K))


# speedup vs baseline: 2.4661x; 1.0839x over previous
import jax
import jax.numpy as jnp
from jax.experimental import pallas as pl
from jax.experimental.pallas import tpu as pltpu

_B, _T, _D = 16, 576, 256
_K = 1024
_N = _B * _T
_BN = 1024


def _body(x_ref, et_ref, e_ref, q_ref, ind_ref, dist_ref):
    x = x_ref[...]
    et = et_ref[...]
    xe = jax.lax.dot_general(
        x, et, (((1,), (0,)), ((), ())),
        preferred_element_type=jnp.float32)
    x2 = jnp.sum(x * x, axis=1, keepdims=True)
    e2 = jnp.sum(et * et, axis=0, keepdims=True)
    dist = -(x2 - 2.0 * xe + e2)
    dist_ref[...] = dist
    m = jnp.max(dist, axis=1, keepdims=True)
    iota_k = jax.lax.broadcasted_iota(jnp.int32, dist.shape, 1)
    ind = jnp.min(jnp.where(dist == m, iota_k, _K), axis=1, keepdims=True)
    ind_ref[...] = ind
    onehot = (iota_k == ind).astype(jnp.float32)
    q_ref[...] = jax.lax.dot_general(
        onehot, e_ref[...], (((1,), (0,)), ((), ())),
        preferred_element_type=jnp.float32)


def kernel(x, x_len, embed):
    del x_len
    xf = x.reshape(_N, _D)
    embed_t = embed.T
    grid = (_N // _BN,)
    q, ind, dist = pl.pallas_call(
        _body,
        grid=grid,
        in_specs=[
            pl.BlockSpec((_BN, _D), lambda i: (i, 0)),
            pl.BlockSpec((_D, _K), lambda i: (0, 0)),
            pl.BlockSpec((_K, _D), lambda i: (0, 0)),
        ],
        out_specs=[
            pl.BlockSpec((_BN, _D), lambda i: (i, 0)),
            pl.BlockSpec((_BN, 1), lambda i: (i, 0)),
            pl.BlockSpec((_BN, _K), lambda i: (i, 0)),
        ],
        out_shape=[
            jax.ShapeDtypeStruct((_N, _D), jnp.float32),
            jax.ShapeDtypeStruct((_N, 1), jnp.int32),
            jax.ShapeDtypeStruct((_N, _K), jnp.float32),
        ],
        compiler_params=pltpu.CompilerParams(
            dimension_semantics=("parallel",),
        ),
    )(xf, embed_t, embed)
    return (q.reshape(_B, _T, _D), ind.reshape(_B, _T), dist.reshape(_B, _T, _K))


# speedup vs baseline: 2.6398x; 1.0704x over previous
import jax
import jax.numpy as jnp
from jax.experimental import pallas as pl
from jax.experimental.pallas import tpu as pltpu

_B, _T, _D = 16, 576, 256
_K = 1024
_N = _B * _T
_BN = 1024


def _body(x_ref, et_hbm, q_ref, ind_ref, dist_ref, et_v, e2_v, sem):
    i = pl.program_id(0)

    @pl.when(i == 0)
    def _init():
        cp = pltpu.make_async_copy(et_hbm, et_v, sem)
        cp.start()
        cp.wait()
        etl = et_v[...]
        e2_v[...] = jnp.sum(etl * etl, axis=0, keepdims=True)

    x = x_ref[...]
    et = et_v[...]
    xe = jax.lax.dot_general(
        x, et, (((1,), (0,)), ((), ())),
        preferred_element_type=jnp.float32)
    x2 = jnp.sum(x * x, axis=1, keepdims=True)
    dist = -(x2 - 2.0 * xe + e2_v[...])
    dist_ref[...] = dist
    m = jnp.max(dist, axis=1, keepdims=True)
    iota_k = jax.lax.broadcasted_iota(jnp.int32, dist.shape, 1)
    ind = jnp.min(jnp.where(dist == m, iota_k, _K), axis=1, keepdims=True)
    ind_ref[...] = ind
    onehot = (iota_k == ind).astype(jnp.float32)
    q_ref[...] = jax.lax.dot_general(
        onehot, et, (((1,), (1,)), ((), ())),
        preferred_element_type=jnp.float32)


def kernel(x, x_len, embed):
    del x_len
    xf = x.reshape(_N, _D)
    embed_t = embed.T
    grid = (_N // _BN,)
    q, ind, dist = pl.pallas_call(
        _body,
        grid=grid,
        in_specs=[
            pl.BlockSpec((_BN, _D), lambda i: (i, 0)),
            pl.BlockSpec(memory_space=pltpu.MemorySpace.HBM),
        ],
        out_specs=[
            pl.BlockSpec((_BN, _D), lambda i: (i, 0)),
            pl.BlockSpec((_BN, 1), lambda i: (i, 0)),
            pl.BlockSpec((_BN, _K), lambda i: (i, 0)),
        ],
        out_shape=[
            jax.ShapeDtypeStruct((_N, _D), jnp.float32),
            jax.ShapeDtypeStruct((_N, 1), jnp.int32),
            jax.ShapeDtypeStruct((_N, _K), jnp.float32),
        ],
        scratch_shapes=[
            pltpu.VMEM((_D, _K), jnp.float32),
            pltpu.VMEM((1, _K), jnp.float32),
            pltpu.SemaphoreType.DMA,
        ],
        compiler_params=pltpu.CompilerParams(
            dimension_semantics=("arbitrary",),
        ),
    )(xf, embed_t)
    return (q.reshape(_B, _T, _D), ind.reshape(_B, _T), dist.reshape(_B, _T, _K))
